# Initial kernel scaffold; baseline (speedup 1.0000x reference)
import functools

import jax
import jax.numpy as jnp
from jax import lax
from jax.experimental import pallas as pl
from jax.experimental.pallas import tpu as pltpu
from jax.experimental.pallas import tpu_sc as plsc

N = 50000
E = 1600000
F = 16
NC = 2
NS = 16
K = 2000
NCH = N // K
EW = E // (NC * NS)
ECH = EW // K
ECH_SC = E // NS // K
BLK = 1000


def _enc_body(x_ref, w_ref, b_ref, a_ref):
    a_ref[...] = (
        jnp.dot(x_ref[...], w_ref[...], preferred_element_type=jnp.float32)
        + b_ref[...]
    )


def _encoder(x, wenc, benc):
    return pl.pallas_call(
        _enc_body,
        grid=(N // BLK,),
        in_specs=[
            pl.BlockSpec((BLK, 630), lambda i: (i, 0)),
            pl.BlockSpec((630, F), lambda i: (0, 0)),
            pl.BlockSpec((1, F), lambda i: (0, 0)),
        ],
        out_specs=pl.BlockSpec((BLK, F), lambda i: (i, 0)),
        out_shape=jax.ShapeDtypeStruct((N, F), jnp.float32),
    )(x, wenc, benc)


def _rsqrt16(v):
    iv = plsc.bitcast(v, jnp.int32)
    y = plsc.bitcast(jnp.int32(0x5F3759DF) - (iv >> 1), jnp.float32)
    for _ in range(3):
        y = y * (1.5 - 0.5 * v * y * y)
    return jnp.where(v > 0.0, y, 0.0)


def _edge_body(ei, a_hbm, t_hbm,
               row_v, col_v, ones_v, deg_v, dis_v, rows_v, zrow_v,
               b_sh, t_sh, deg_sh, sem):
    c = lax.axis_index("c")
    s = lax.axis_index("s")
    w = c * NS + s

    def _fill1(i, _):
        ones_v[pl.ds(i * 16, 16)] = jnp.full((16,), 1.0, jnp.float32)
        deg_v[pl.ds(i * 16, 16)] = jnp.zeros((16,), jnp.float32)
        return 0
    lax.fori_loop(0, K // 16, _fill1, 0)

    def _fill2(i, _):
        zrow_v[i] = jnp.zeros((F,), jnp.float32)
        return 0
    lax.fori_loop(0, K, _fill2, 0)

    for k in range(2):
        ch = s + NS * k
        @pl.when(ch < NCH)
        def _():
            base = ch * K
            pltpu.sync_copy(zrow_v, t_sh.at[pl.ds(base, K)])
            pltpu.sync_copy(deg_v, deg_sh.at[pl.ds(base, K)])
    plsc.subcore_barrier()

    def _deg_step(i, _):
        base = s * (E // NS) + i * K
        pltpu.sync_copy(ei.at[0, pl.ds(base, K)], row_v)
        pltpu.sync_copy(ones_v, deg_sh.at[row_v], add=True)
        return 0
    lax.fori_loop(0, ECH_SC, _deg_step, 0)
    plsc.subcore_barrier()

    for k in range(2):
        ch = s + NS * k
        @pl.when(ch < NCH)
        def _():
            base = ch * K
            pltpu.sync_copy(deg_sh.at[pl.ds(base, K)], deg_v)

            def _dis(j, _):
                v = deg_v[pl.ds(j * 16, 16)]
                dis_v[k, pl.ds(j * 16, 16)] = _rsqrt16(v)
                return 0
            lax.fori_loop(0, K // 16, _dis, 0)

            pltpu.sync_copy(a_hbm.at[pl.ds(base, K)], rows_v)

            def _scale(i, _):
                rows_v[i] = rows_v[i] * dis_v[k, i]
                return 0
            lax.fori_loop(0, K, _scale, 0)
            pltpu.sync_copy(rows_v, b_sh.at[pl.ds(base, K)])
    plsc.subcore_barrier()

    def _edge_step(i, _):
        base = w * EW + i * K
        pltpu.sync_copy(ei.at[0, pl.ds(base, K)], row_v)
        pltpu.sync_copy(ei.at[1, pl.ds(base, K)], col_v)
        pltpu.async_copy(b_sh.at[row_v], rows_v, sem).wait()
        pltpu.sync_copy(rows_v, t_sh.at[col_v], add=True)
        return 0
    lax.fori_loop(0, ECH, _edge_step, 0)
    plsc.subcore_barrier()

    for k in range(2):
        ch = s + NS * k
        @pl.when(ch < NCH)
        def _():
            base = ch * K
            pltpu.sync_copy(t_sh.at[pl.ds(base, K)], rows_v)

            def _scale(i, _):
                rows_v[i] = rows_v[i] * (0.0 - dis_v[k, i])
                return 0
            lax.fori_loop(0, K, _scale, 0)
            pltpu.sync_copy(rows_v, t_hbm.at[c, pl.ds(base, K)])


def _edge_sc(ei, a):
    mesh = plsc.VectorSubcoreMesh(core_axis_name="c", subcore_axis_name="s")
    fn = functools.partial(
        pl.kernel,
        out_type=jax.ShapeDtypeStruct((NC, N, F), jnp.float32),
        mesh=mesh,
        scratch_types=[
            pltpu.VMEM((K,), jnp.int32),
            pltpu.VMEM((K,), jnp.int32),
            pltpu.VMEM((K,), jnp.float32),
            pltpu.VMEM((K,), jnp.float32),
            pltpu.VMEM((2, K), jnp.float32),
            pltpu.VMEM((K, F), jnp.float32),
            pltpu.VMEM((K, F), jnp.float32),
            pltpu.VMEM_SHARED((N, F), jnp.float32),
            pltpu.VMEM_SHARED((N, F), jnp.float32),
            pltpu.VMEM_SHARED((N,), jnp.float32),
            pltpu.SemaphoreType.DMA,
        ],
    )(_edge_body)
    return fn(ei, a)


def _fin_body(a_ref, t_ref, wza_ref, wzs_ref, cz_ref, wha_ref, whs_ref,
              ch_ref, wl_ref, bl_ref, o_ref):
    a = a_ref[...]
    sm = t_ref[0] + t_ref[1]
    z = jax.nn.sigmoid(
        jnp.dot(a, wza_ref[...], preferred_element_type=jnp.float32)
        + jnp.dot(sm, wzs_ref[...], preferred_element_type=jnp.float32)
        + cz_ref[...]
    )
    ht = jnp.tanh(
        jnp.dot(a, wha_ref[...], preferred_element_type=jnp.float32)
        + jnp.dot(sm, whs_ref[...], preferred_element_type=jnp.float32)
        + ch_ref[...]
    )
    h = jax.nn.relu((1.0 - z) * ht)
    o_ref[...] = jax.nn.sigmoid(
        jnp.dot(h, wl_ref[...], preferred_element_type=jnp.float32)
        + bl_ref[...]
    )


def _finish(a, t, wza, wzs, cz, wha, whs, chb, wl, bl):
    return pl.pallas_call(
        _fin_body,
        grid=(N // BLK,),
        in_specs=[
            pl.BlockSpec((BLK, F), lambda i: (i, 0)),
            pl.BlockSpec((NC, BLK, F), lambda i: (0, i, 0)),
            pl.BlockSpec((F, 64), lambda i: (0, 0)),
            pl.BlockSpec((F, 64), lambda i: (0, 0)),
            pl.BlockSpec((1, 64), lambda i: (0, 0)),
            pl.BlockSpec((F, 64), lambda i: (0, 0)),
            pl.BlockSpec((F, 64), lambda i: (0, 0)),
            pl.BlockSpec((1, 64), lambda i: (0, 0)),
            pl.BlockSpec((64, 1), lambda i: (0, 0)),
            pl.BlockSpec((1, 1), lambda i: (0, 0)),
        ],
        out_specs=pl.BlockSpec((BLK, 1), lambda i: (i, 0)),
        out_shape=jax.ShapeDtypeStruct((N, 1), jnp.float32),
    )(a, t, wza, wzs, cz, wha, whs, chb, wl, bl)


def kernel(x, edge_index, We, be, Wxz, bxz, Whz, bhz, Wxr, bxr, Whr, bhr,
           Wxh, bxh, Whh, bhh, Wl, bl):
    f32 = jnp.float32
    wenc = jnp.zeros((630, F), f32)
    wenc = wenc.at[0:314, 0:4].set(We)
    wenc = wenc.at[314, 4].set(1.0)
    wenc = wenc.at[315:629, 5:9].set(We)
    wenc = wenc.at[629, 9].set(1.0)
    benc = jnp.zeros((1, F), f32)
    benc = benc.at[0, 0:4].set(be)
    benc = benc.at[0, 5:9].set(be)

    wza = jnp.zeros((F, 64), f32).at[0:10, :].set(Wxz[0])
    wzs = jnp.zeros((F, 64), f32).at[0:10, :].set(Wxz[1])
    cz = (bxz + bhz).reshape(1, 64)
    wha = jnp.zeros((F, 64), f32).at[0:10, :].set(Wxh[0])
    whs = jnp.zeros((F, 64), f32).at[0:10, :].set(Wxh[1])
    chb = (bxh + bhh).reshape(1, 64)

    a = _encoder(x, wenc, benc)
    t = _edge_sc(edge_index, a)
    return _finish(a, t, wza, wzs, cz, wha, whs, chb,
                   Wl.reshape(64, 1), bl.reshape(1, 1))


# speedup vs baseline: 77.0054x; 77.0054x over previous
import functools

import jax
import jax.numpy as jnp
from jax import lax
from jax.experimental import pallas as pl
from jax.experimental.pallas import tpu as pltpu
from jax.experimental.pallas import tpu_sc as plsc

N = 50000
E = 1600000
F = 16
NC = 2
NS = 16
K = 2000
NCH = N // K
EW = E // (NC * NS)
ECH = EW // K
ECH_SC = E // NS // K
BLK = 1000


def _enc_body(x_ref, w_ref, b_ref, a_ref):
    a_ref[...] = (
        jnp.dot(x_ref[...], w_ref[...], preferred_element_type=jnp.float32)
        + b_ref[...]
    )


def _encoder(x, wenc, benc):
    return pl.pallas_call(
        _enc_body,
        grid=(N // BLK,),
        in_specs=[
            pl.BlockSpec((BLK, 630), lambda i: (i, 0)),
            pl.BlockSpec((630, F), lambda i: (0, 0)),
            pl.BlockSpec((1, F), lambda i: (0, 0)),
        ],
        out_specs=pl.BlockSpec((BLK, F), lambda i: (i, 0)),
        out_shape=jax.ShapeDtypeStruct((N, F), jnp.float32),
    )(x, wenc, benc)


def _rsqrt16(v):
    iv = lax.bitcast_convert_type(v, jnp.int32)
    y = lax.bitcast_convert_type(jnp.int32(0x5F3759DF) - (iv >> 1), jnp.float32)
    for _ in range(3):
        y = y * (1.5 - 0.5 * v * y * y)
    return jnp.where(v > 0.0, y, 0.0)


def _edge_body(ei, a_hbm, t_hbm, b_hbm,
               row_v, col_v, ones_v, deg_v, dis_v, rows_v, zrow_v,
               t_sh, deg_sh, sem):
    c = lax.axis_index("c")
    s = lax.axis_index("s")
    w = c * NS + s

    def _fill1(i, _):
        ones_v[pl.ds(i * 16, 16)] = jnp.full((16,), 1.0, jnp.float32)
        deg_v[pl.ds(i * 16, 16)] = jnp.zeros((16,), jnp.float32)
        return 0
    lax.fori_loop(0, K // 16, _fill1, 0)

    def _fill2(i, _):
        zrow_v[i] = jnp.zeros((F,), jnp.float32)
        return 0
    lax.fori_loop(0, K, _fill2, 0)

    for k in range(2):
        ch = s + NS * k
        @pl.when(ch < NCH)
        def _():
            base = ch * K
            pltpu.sync_copy(zrow_v, t_sh.at[pl.ds(base, K)])
            pltpu.sync_copy(deg_v, deg_sh.at[pl.ds(base, K)])
    plsc.subcore_barrier()

    def _deg_step(i, _):
        base = s * (E // NS) + i * K
        pltpu.sync_copy(ei.at[pl.ds(base, K)], row_v)
        pltpu.sync_copy(ones_v, deg_sh.at[row_v], add=True)
        return 0
    lax.fori_loop(0, ECH_SC, _deg_step, 0)
    plsc.subcore_barrier()

    for k in range(2):
        ch = s + NS * k
        @pl.when(ch < NCH)
        def _():
            base = ch * K
            pltpu.sync_copy(deg_sh.at[pl.ds(base, K)], deg_v)

            def _dis(j, _):
                v = deg_v[pl.ds(j * 16, 16)]
                dis_v[k, pl.ds(j * 16, 16)] = _rsqrt16(v)
                return 0
            lax.fori_loop(0, K // 16, _dis, 0)

            pltpu.sync_copy(a_hbm.at[pl.ds(base, K)], rows_v)

            def _scale(j, _):
                d = dis_v[k, pl.ds(j * 16, 16)]
                for l in range(16):
                    i = j * 16 + l
                    rows_v[i] = rows_v[i] * d[l]
                return 0
            lax.fori_loop(0, K // 16, _scale, 0)
            pltpu.sync_copy(rows_v, b_hbm.at[c, pl.ds(base, K)])
    plsc.subcore_barrier()

    def _edge_step(i, _):
        base = w * EW + i * K
        pltpu.sync_copy(ei.at[pl.ds(base, K)], row_v)
        pltpu.sync_copy(ei.at[pl.ds(E + base, K)], col_v)
        pltpu.async_copy(b_hbm.at[c].at[row_v], rows_v, sem).wait()
        pltpu.sync_copy(rows_v, t_sh.at[col_v], add=True)
        return 0
    lax.fori_loop(0, ECH, _edge_step, 0)
    plsc.subcore_barrier()

    for k in range(2):
        ch = s + NS * k
        @pl.when(ch < NCH)
        def _():
            base = ch * K
            pltpu.sync_copy(t_sh.at[pl.ds(base, K)], rows_v)

            def _scale(j, _):
                d = dis_v[k, pl.ds(j * 16, 16)]
                for l in range(16):
                    i = j * 16 + l
                    rows_v[i] = rows_v[i] * (0.0 - d[l])
                return 0
            lax.fori_loop(0, K // 16, _scale, 0)
            pltpu.sync_copy(rows_v, t_hbm.at[c, pl.ds(base, K)])


def _edge_sc(ei, a):
    mesh = plsc.VectorSubcoreMesh(core_axis_name="c", subcore_axis_name="s")
    fn = functools.partial(
        pl.kernel,
        out_type=[
            jax.ShapeDtypeStruct((NC, N, F), jnp.float32),
            jax.ShapeDtypeStruct((NC, N, F), jnp.float32),
        ],
        mesh=mesh,
        scratch_types=[
            pltpu.VMEM((K,), jnp.int32),
            pltpu.VMEM((K,), jnp.int32),
            pltpu.VMEM((K,), jnp.float32),
            pltpu.VMEM((K,), jnp.float32),
            pltpu.VMEM((2, K), jnp.float32),
            pltpu.VMEM((K, F), jnp.float32),
            pltpu.VMEM((K, F), jnp.float32),
            pltpu.VMEM_SHARED((N, F), jnp.float32),
            pltpu.VMEM_SHARED((N,), jnp.float32),
            pltpu.SemaphoreType.DMA,
        ],
        compiler_params=pltpu.CompilerParams(use_tc_tiling_on_sc=False),
    )(_edge_body)
    t, _b = fn(ei.reshape(-1), a)
    return t


def _fin_body(a_ref, t_ref, wza_ref, wzs_ref, cz_ref, wha_ref, whs_ref,
              ch_ref, wl_ref, bl_ref, o_ref):
    a = a_ref[...]
    sm = t_ref[0] + t_ref[1]
    z = jax.nn.sigmoid(
        jnp.dot(a, wza_ref[...], preferred_element_type=jnp.float32)
        + jnp.dot(sm, wzs_ref[...], preferred_element_type=jnp.float32)
        + cz_ref[...]
    )
    ht = jnp.tanh(
        jnp.dot(a, wha_ref[...], preferred_element_type=jnp.float32)
        + jnp.dot(sm, whs_ref[...], preferred_element_type=jnp.float32)
        + ch_ref[...]
    )
    h = jax.nn.relu((1.0 - z) * ht)
    o_ref[...] = jax.nn.sigmoid(
        jnp.dot(h, wl_ref[...], preferred_element_type=jnp.float32)
        + bl_ref[...]
    )


def _finish(a, t, wza, wzs, cz, wha, whs, chb, wl, bl):
    return pl.pallas_call(
        _fin_body,
        grid=(N // BLK,),
        in_specs=[
            pl.BlockSpec((BLK, F), lambda i: (i, 0)),
            pl.BlockSpec((NC, BLK, F), lambda i: (0, i, 0)),
            pl.BlockSpec((F, 64), lambda i: (0, 0)),
            pl.BlockSpec((F, 64), lambda i: (0, 0)),
            pl.BlockSpec((1, 64), lambda i: (0, 0)),
            pl.BlockSpec((F, 64), lambda i: (0, 0)),
            pl.BlockSpec((F, 64), lambda i: (0, 0)),
            pl.BlockSpec((1, 64), lambda i: (0, 0)),
            pl.BlockSpec((64, 1), lambda i: (0, 0)),
            pl.BlockSpec((1, 1), lambda i: (0, 0)),
        ],
        out_specs=pl.BlockSpec((BLK, 1), lambda i: (i, 0)),
        out_shape=jax.ShapeDtypeStruct((N, 1), jnp.float32),
    )(a, t, wza, wzs, cz, wha, whs, chb, wl, bl)


def kernel(x, edge_index, We, be, Wxz, bxz, Whz, bhz, Wxr, bxr, Whr, bhr,
           Wxh, bxh, Whh, bhh, Wl, bl):
    f32 = jnp.float32
    wenc = jnp.zeros((630, F), f32)
    wenc = wenc.at[0:314, 0:4].set(We)
    wenc = wenc.at[314, 4].set(1.0)
    wenc = wenc.at[315:629, 5:9].set(We)
    wenc = wenc.at[629, 9].set(1.0)
    benc = jnp.zeros((1, F), f32)
    benc = benc.at[0, 0:4].set(be)
    benc = benc.at[0, 5:9].set(be)

    wza = jnp.zeros((F, 64), f32).at[0:10, :].set(Wxz[0])
    wzs = jnp.zeros((F, 64), f32).at[0:10, :].set(Wxz[1])
    cz = (bxz + bhz).reshape(1, 64)
    wha = jnp.zeros((F, 64), f32).at[0:10, :].set(Wxh[0])
    whs = jnp.zeros((F, 64), f32).at[0:10, :].set(Wxh[1])
    chb = (bxh + bhh).reshape(1, 64)

    a = _encoder(x, wenc, benc)
    t = _edge_sc(edge_index, a)
    return _finish(a, t, wza, wzs, cz, wha, whs, chb,
                   Wl.reshape(64, 1), bl.reshape(1, 1))
